# baseline (speedup 1.0000x reference)
import jax
import jax.numpy as jnp
from jax.experimental import pallas as pl
from jax.experimental.pallas import tpu as pltpu

D_MODEL = 2048
D_FF = 768
BLK_T = 1024


def _mlp_block(x_ref, wgu_ref, wd_ref, o_ref):
    d_ff = wd_ref.shape[0]
    xb = x_ref[...].astype(jnp.bfloat16)
    gu = jnp.dot(xb, wgu_ref[...], preferred_element_type=jnp.float32)
    gate = gu[:, :d_ff]
    up = gu[:, d_ff:]
    hidden = (jax.nn.silu(gate) * up).astype(jnp.bfloat16)
    o_ref[...] = jnp.dot(hidden, wd_ref[...], preferred_element_type=jnp.float32)


def kernel(x, W_gate, W_up, W_down):
    n_tokens, d_model = x.shape
    d_ff = W_gate.shape[1]
    wgu = jnp.concatenate(
        [W_gate.astype(jnp.bfloat16), W_up.astype(jnp.bfloat16)], axis=1)
    wd = W_down.astype(jnp.bfloat16)
    grid = (n_tokens // BLK_T,)
    return pl.pallas_call(
        _mlp_block,
        grid=grid,
        in_specs=[
            pl.BlockSpec((BLK_T, d_model), lambda i: (i, 0)),
            pl.BlockSpec((d_model, 2 * d_ff), lambda i: (0, 0)),
            pl.BlockSpec((d_ff, d_model), lambda i: (0, 0)),
        ],
        out_specs=pl.BlockSpec((BLK_T, d_model), lambda i: (i, 0)),
        out_shape=jax.ShapeDtypeStruct((n_tokens, d_model), jnp.float32),
        compiler_params=pltpu.CompilerParams(
            dimension_semantics=("parallel",),
        ),
    )(x, wgu, wd)


# speedup vs baseline: 1.0060x; 1.0060x over previous
import jax
import jax.numpy as jnp
from jax.experimental import pallas as pl
from jax.experimental.pallas import tpu as pltpu

BLK_T = 512


def _mlp_block(x_ref, wg_ref, wu_ref, wd_ref, o_ref, wgu_s, wd_s):
    d_ff = wd_ref.shape[0]

    @pl.when(pl.program_id(0) == 0)
    def _cast_weights():
        wgu_s[:, :d_ff] = wg_ref[...].astype(jnp.bfloat16)
        wgu_s[:, d_ff:] = wu_ref[...].astype(jnp.bfloat16)
        wd_s[...] = wd_ref[...].astype(jnp.bfloat16)

    xb = x_ref[...].astype(jnp.bfloat16)
    gu = jnp.dot(xb, wgu_s[...], preferred_element_type=jnp.float32)
    gate = gu[:, :d_ff]
    up = gu[:, d_ff:]
    hidden = (jax.nn.silu(gate) * up).astype(jnp.bfloat16)
    o_ref[...] = jnp.dot(hidden, wd_s[...], preferred_element_type=jnp.float32)


def kernel(x, W_gate, W_up, W_down):
    n_tokens, d_model = x.shape
    d_ff = W_gate.shape[1]
    grid = (n_tokens // BLK_T,)
    return pl.pallas_call(
        _mlp_block,
        grid=grid,
        in_specs=[
            pl.BlockSpec((BLK_T, d_model), lambda i: (i, 0)),
            pl.BlockSpec((d_model, d_ff), lambda i: (0, 0)),
            pl.BlockSpec((d_model, d_ff), lambda i: (0, 0)),
            pl.BlockSpec((d_ff, d_model), lambda i: (0, 0)),
        ],
        out_specs=pl.BlockSpec((BLK_T, d_model), lambda i: (i, 0)),
        out_shape=jax.ShapeDtypeStruct((n_tokens, d_model), jnp.float32),
        scratch_shapes=[
            pltpu.VMEM((d_model, 2 * d_ff), jnp.bfloat16),
            pltpu.VMEM((d_ff, d_model), jnp.bfloat16),
        ],
        compiler_params=pltpu.CompilerParams(
            dimension_semantics=("arbitrary",),
        ),
    )(x, W_gate, W_up, W_down)


# speedup vs baseline: 1.0270x; 1.0209x over previous
import jax
import jax.numpy as jnp
from jax.experimental import pallas as pl
from jax.experimental.pallas import tpu as pltpu

BLK_T = 1024


def _mlp_block(x_ref, wg_ref, wu_ref, wd_ref, o_ref):
    xb = x_ref[...]
    gate = jnp.dot(xb, wg_ref[...], preferred_element_type=jnp.float32)
    up = jnp.dot(xb, wu_ref[...], preferred_element_type=jnp.float32)
    hidden = jax.nn.silu(gate) * up
    o_ref[...] = jnp.dot(hidden, wd_ref[...], preferred_element_type=jnp.float32)


def kernel(x, W_gate, W_up, W_down):
    n_tokens, d_model = x.shape
    d_ff = W_gate.shape[1]
    grid = (n_tokens // BLK_T,)
    return pl.pallas_call(
        _mlp_block,
        grid=grid,
        in_specs=[
            pl.BlockSpec((BLK_T, d_model), lambda i: (i, 0)),
            pl.BlockSpec((d_model, d_ff), lambda i: (0, 0)),
            pl.BlockSpec((d_model, d_ff), lambda i: (0, 0)),
            pl.BlockSpec((d_ff, d_model), lambda i: (0, 0)),
        ],
        out_specs=pl.BlockSpec((BLK_T, d_model), lambda i: (i, 0)),
        out_shape=jax.ShapeDtypeStruct((n_tokens, d_model), jnp.float32),
        compiler_params=pltpu.CompilerParams(
            dimension_semantics=("arbitrary",),
        ),
    )(x, W_gate, W_up, W_down)
